# baseline (speedup 1.0000x reference)
import jax
import jax.numpy as jnp
from jax import lax
from jax.experimental import pallas as pl
from jax.experimental.pallas import tpu as pltpu
from jax.experimental.pallas import tpu_sc as plsc

B = 16384
L = 20
D = 64
NC = 2
NS = 16
NW = NC * NS
BPW = B // NW
SUB = 32
NSUB = BPW // SUB
CPS = SUB * L
BG = 4
IDXW = 128


def _perm(v, idx):
    return jnp.take_along_axis(v, idx, axis=0, mode="promise_in_bounds")


def _tree_reduce16(accs, perms, masks, brev):
    vs = accs
    for s, d in enumerate((8, 4, 2, 1)):
        m, p = masks[s], perms[s]
        vs = [jnp.where(m, vs[2 * i], vs[2 * i + 1])
              + _perm(jnp.where(m, vs[2 * i + 1], vs[2 * i]), p)
              for i in range(len(vs) // 2)]
    return _perm(vs[0], brev)


def _sc_body(word_ids_r, ctx_ids_r, word_embed, context_embed, out_hbm,
             idx_w, idx_c, w_rows, c_rows, out_v, sem):
    wid = lax.axis_index("c") * NS + lax.axis_index("s")

    lane = lax.iota(jnp.int32, 16)
    perms = []
    masks = []
    for d in (8, 4, 2, 1):
        perms.append((lane & ~(2 * d - 1)) | ((lane + d) & (2 * d - 1)))
        masks.append((lane % (2 * d)) < d)
    brev = (((lane & 1) << 3) | ((lane & 2) << 1)
            | ((lane & 4) >> 1) | ((lane & 8) >> 3))

    pltpu.sync_copy(word_ids_r.at[pl.ds(wid * 4, 4)], idx_w)
    pltpu.sync_copy(ctx_ids_r.at[pl.ds(wid * 80, 80)], idx_c)

    w_copies = []
    for j in range(4):
        w_copies.append(pltpu.async_copy(
            word_embed.at[idx_w.at[j]],
            w_rows.at[pl.ds(j * IDXW, IDXW)], sem))
    for c in w_copies:
        c.wait()

    def chunk_body(sub, _):
        c_copies = []
        for j in range(5):
            c_copies.append(pltpu.async_copy(
                context_embed.at[idx_c.at[sub * 5 + j]],
                c_rows.at[pl.ds(j * IDXW, IDXW)], sem))
        for c in c_copies:
            c.wait()

        def group_body(bg, _):
            row0 = sub * SUB + bg * BG
            wv = [[w_rows[row0 + bi, pl.ds(k * 16, 16)] for k in range(4)]
                  for bi in range(BG)]
            cbase = bg * (BG * L)
            for g in range(5):
                accs = []
                for o in range(16):
                    f = g * 16 + o
                    cr = cbase + f
                    bi = f // L
                    acc = wv[bi][0] * c_rows[cr, pl.ds(0, 16)]
                    for k in range(1, 4):
                        acc = acc + wv[bi][k] * c_rows[cr, pl.ds(k * 16, 16)]
                    accs.append(acc)
                res = _tree_reduce16(accs, perms, masks, brev)
                out_v[pl.ds(sub * (SUB * L) + cbase + g * 16, 16)] = res
            return ()

        lax.fori_loop(0, SUB // BG, group_body, (), unroll=False)
        return ()

    lax.fori_loop(0, NSUB, chunk_body, (), unroll=False)

    pltpu.sync_copy(out_v, out_hbm.at[pl.ds(wid * BPW * L, BPW * L)])


@jax.jit
def _word2vec_sc(word_ids_r, ctx_ids_r, word_embed, context_embed):
    mesh = plsc.VectorSubcoreMesh(core_axis_name="c", subcore_axis_name="s")
    return pl.kernel(
        _sc_body,
        out_type=jax.ShapeDtypeStruct((B * L,), jnp.float32),
        mesh=mesh,
        compiler_params=pltpu.CompilerParams(use_tc_tiling_on_sc=False),
        scratch_types=[
            pltpu.VMEM((4, IDXW), jnp.int32),
            pltpu.VMEM((80, IDXW), jnp.int32),
            pltpu.VMEM((BPW, D), jnp.float32),
            pltpu.VMEM((CPS, D), jnp.float32),
            pltpu.VMEM((BPW * L,), jnp.float32),
            pltpu.SemaphoreType.DMA,
        ],
    )(word_ids_r, ctx_ids_r, word_embed, context_embed)


def kernel(word_ids, context_ids, word_embed, context_embed):
    word_ids_r = word_ids.reshape(B // IDXW, IDXW)
    ctx_ids_r = context_ids.reshape(B * L // IDXW, IDXW)
    return _word2vec_sc(word_ids_r, ctx_ids_r, word_embed, context_embed).reshape(B, L)
